# Initial kernel scaffold; baseline (speedup 1.0000x reference)
import jax
import jax.numpy as jnp
from jax.experimental import pallas as pl


def kernel(flat, row_lengths):
    raise NotImplementedError("write your pallas kernel here")


# speedup vs baseline: 1.4397x; 1.4397x over previous
import functools

import jax
import jax.numpy as jnp
from jax import lax
from jax.experimental import pallas as pl
from jax.experimental.pallas import tpu as pltpu
from jax.experimental.pallas import tpu_sc as plsc

_LANES = 16


@functools.lru_cache(maxsize=None)
def _make_linkage_kernel(total: int, bsz: int):
    info = plsc.get_sparse_core_info()
    nw = info.num_cores * info.num_subcores
    num_link = total - bsz
    flat_n = 4 * num_link
    assert flat_n % _LANES == 0
    ch = -(-flat_n // nw)
    ch = -(-ch // _LANES) * _LANES
    last = flat_n - (nw - 1) * ch
    assert 0 < last <= ch and last % _LANES == 0 and ch % 8 == 0
    rows = ch // _LANES

    mesh = plsc.VectorSubcoreMesh(core_axis_name="c", subcore_axis_name="s")

    @functools.partial(
        pl.kernel,
        mesh=mesh,
        out_type=jax.ShapeDtypeStruct((flat_n,), jnp.int32),
        scratch_types=[
            pltpu.VMEM((_LANES,), jnp.int32),
            pltpu.VMEM((ch,), jnp.int32),
        ],
    )
    def k(rl_hbm, out_hbm, rl_v, buf):
        wid = lax.axis_index("s") * info.num_cores + lax.axis_index("c")

        pltpu.sync_copy(rl_hbm, rl_v)
        rl_vec = rl_v[...]
        thr = []
        run = None
        for t in range(bsz - 1):
            rl_b = rl_vec.at[jnp.full((_LANES,), t, jnp.int32)].get(
                mode="promise_in_bounds")
            run = (rl_b - 1) if run is None else run + (rl_b - 1)
            thr.append(run)

        iota = lax.iota(jnp.int32, _LANES)
        lane_i = iota >> 2
        delta = ((iota & 3) ^ ((iota & 3) >> 1)) & 1
        base_vec = lane_i + delta + (bsz - 1)

        row0 = wid * rows

        def body(r, carry):
            g4 = (row0 + r) * 4
            i_vec = lane_i + g4
            acc = base_vec + g4
            for t in thr:
                acc = acc + ((i_vec - t) >> 31)
            buf[pl.ds(r * _LANES, _LANES)] = acc
            return carry

        lax.fori_loop(0, rows, body, 0)

        @pl.when(wid < nw - 1)
        def _():
            pltpu.sync_copy(buf, out_hbm.at[pl.ds(wid * ch, ch)])

        @pl.when(wid == nw - 1)
        def _():
            pltpu.sync_copy(buf.at[pl.ds(0, last)],
                            out_hbm.at[pl.ds((nw - 1) * ch, last)])

    return k


def kernel(flat, row_lengths):
    total = flat.shape[0]
    bsz = row_lengths.shape[0]
    link_flat = _make_linkage_kernel(total, bsz)(row_lengths)
    return flat, link_flat.reshape(-1, 2)


# speedup vs baseline: 3.0809x; 2.1399x over previous
import functools

import jax
import jax.numpy as jnp
from jax import lax
from jax.experimental import pallas as pl
from jax.experimental.pallas import tpu as pltpu
from jax.experimental.pallas import tpu_sc as plsc

_LANES = 16


@functools.lru_cache(maxsize=None)
def _make_token_index_kernel(total: int, bsz: int):
    info = plsc.get_sparse_core_info()
    nw = info.num_cores * info.num_subcores
    n_edges = 2 * (total - bsz)
    assert n_edges % _LANES == 0
    ch = -(-n_edges // nw)
    ch = -(-ch // _LANES) * _LANES
    last = n_edges - (nw - 1) * ch
    assert 0 < last <= ch and last % _LANES == 0 and ch % 8 == 0
    rows = ch // _LANES

    mesh = plsc.VectorSubcoreMesh(core_axis_name="c", subcore_axis_name="s")

    @functools.partial(
        pl.kernel,
        mesh=mesh,
        out_type=jax.ShapeDtypeStruct((n_edges,), jnp.int32),
        scratch_types=[
            pltpu.VMEM((_LANES,), jnp.int32),
            pltpu.VMEM((ch,), jnp.int32),
        ],
    )
    def k(rl_hbm, out_hbm, rl_v, buf):
        wid = lax.axis_index("s") * info.num_cores + lax.axis_index("c")

        pltpu.sync_copy(rl_hbm, rl_v)
        rl_vec = rl_v[...]
        thr = []
        run = None
        for t in range(bsz - 1):
            rl_b = rl_vec.at[jnp.full((_LANES,), t, jnp.int32)].get(
                mode="promise_in_bounds")
            run = (rl_b - 1) if run is None else run + (rl_b - 1)
            thr.append(run)

        iota = lax.iota(jnp.int32, _LANES)
        lane_i = iota >> 1
        base_vec = lane_i + (bsz - 1)

        e0 = wid * ch

        def body(r, carry):
            g = (e0 + r * _LANES) >> 1
            i_vec = lane_i + g
            acc = base_vec + g
            for t in thr:
                acc = acc + ((i_vec - t) >> 31)
            buf[pl.ds(r * _LANES, _LANES)] = acc
            return carry

        lax.fori_loop(0, rows, body, 0)

        @pl.when(wid < nw - 1)
        def _():
            pltpu.sync_copy(buf, out_hbm.at[pl.ds(wid * ch, ch)])

        @pl.when(wid == nw - 1)
        def _():
            pltpu.sync_copy(buf.at[pl.ds(0, last)],
                            out_hbm.at[pl.ds((nw - 1) * ch, last)])

    return k


def kernel(flat, row_lengths):
    total = flat.shape[0]
    bsz = row_lengths.shape[0]
    n_edges = 2 * (total - bsz)
    tok = _make_token_index_kernel(total, bsz)(row_lengths)
    par = lax.iota(jnp.int32, n_edges) & 1
    linkages = jnp.stack([tok + par, tok + (1 - par)], axis=1)
    return flat, linkages
